# baseline (speedup 1.0000x reference)
import functools

import jax
import jax.numpy as jnp
from jax import lax
from jax.experimental import pallas as pl
from jax.experimental.pallas import tpu as pltpu
from jax.experimental.pallas import tpu_sc as plsc

EMB = 128
B_BLOCK = 2048


def _gather_chunk(tab_v, idx_ref, off, buf, chunk):
    def g_body(g, carry):
        rvec = idx_ref[pl.ds(off + g * 16, 16)]
        for j in range(16):
            rid = rvec[j]
            r = g * 16 + j
            for c in range(EMB // 16):
                buf[r, pl.ds(c * 16, 16)] = tab_v[rid, pl.ds(c * 16, 16)]
        return carry
    lax.fori_loop(0, chunk // 16, g_body, 0)


def _sc_gather_body(m_tab_hbm, d_tab_hbm, mi_hbm, di_hbm, om_hbm, od_hbm,
                    m_tab_v, d_tab_v, idx_m, idx_d, buf_a, buf_b, s_a, s_b,
                    *, n_cores, b_per_w):
    chunk = b_per_w // 2
    wid = lax.axis_index("s") * n_cores + lax.axis_index("c")
    base = wid * b_per_w
    pltpu.sync_copy(m_tab_hbm, m_tab_v)
    pltpu.sync_copy(d_tab_hbm, d_tab_v)
    pltpu.sync_copy(mi_hbm.at[pl.ds(base, b_per_w)], idx_m)
    pltpu.sync_copy(di_hbm.at[pl.ds(base, b_per_w)], idx_d)
    _gather_chunk(m_tab_v, idx_m, 0, buf_a, chunk)
    wb_a = pltpu.async_copy(buf_a, om_hbm.at[pl.ds(base, chunk)], s_a)
    _gather_chunk(m_tab_v, idx_m, chunk, buf_b, chunk)
    wb_b = pltpu.async_copy(buf_b, om_hbm.at[pl.ds(base + chunk, chunk)], s_b)
    wb_a.wait()
    _gather_chunk(d_tab_v, idx_d, 0, buf_a, chunk)
    wb_a = pltpu.async_copy(buf_a, od_hbm.at[pl.ds(base, chunk)], s_a)
    wb_b.wait()
    _gather_chunk(d_tab_v, idx_d, chunk, buf_b, chunk)
    wb_b = pltpu.async_copy(buf_b, od_hbm.at[pl.ds(base + chunk, chunk)], s_b)
    wb_a.wait()
    wb_b.wait()


def _sc_gather(model_emb_w, decoding_emb_w, model_ids, decoding_ids):
    B = model_ids.shape[0]
    info = plsc.get_sparse_core_info()
    nw = info.num_cores * info.num_subcores
    b_per_w = B // nw
    mesh = plsc.VectorSubcoreMesh(core_axis_name="c", subcore_axis_name="s")
    k = pl.kernel(
        functools.partial(_sc_gather_body, n_cores=info.num_cores,
                          b_per_w=b_per_w),
        mesh=mesh,
        compiler_params=pltpu.CompilerParams(needs_layout_passes=False),
        out_type=[
            jax.ShapeDtypeStruct((B, EMB), jnp.float32),
            jax.ShapeDtypeStruct((B, EMB), jnp.float32),
        ],
        scratch_types=[
            pltpu.VMEM(model_emb_w.shape, jnp.float32),
            pltpu.VMEM(decoding_emb_w.shape, jnp.float32),
            pltpu.VMEM((b_per_w,), jnp.int32),
            pltpu.VMEM((b_per_w,), jnp.int32),
            pltpu.VMEM((b_per_w // 2, EMB), jnp.float32),
            pltpu.VMEM((b_per_w // 2, EMB), jnp.float32),
            pltpu.SemaphoreType.DMA,
            pltpu.SemaphoreType.DMA,
        ],
    )
    return k(model_emb_w, decoding_emb_w, model_ids, decoding_ids)


def _mlp_body(x_ref, m_ref, d_ref, qw_ref, qb_ref, w1_ref, b1_ref, w2_ref,
              b2_ref, out_ref, hcat_ref, wcat_ref):
    bb = hcat_ref.shape[0]
    bf = jnp.bfloat16

    @pl.when(pl.program_id(0) == 0)
    def _fold():
        wcat_ref[:, : 2 * EMB] = w1_ref[:, : 2 * EMB].astype(bf)
        qfold = lax.dot_general(
            w1_ref[:, 5 * EMB: 6 * EMB].astype(bf), qw_ref[...],
            (((1,), (0,)), ((), ())), preferred_element_type=jnp.float32)
        wcat_ref[:, 2 * EMB: 3 * EMB] = (
            w1_ref[:, 2 * EMB: 3 * EMB] + qfold).astype(bf)
        wcat_ref[:, 3 * EMB: 5 * EMB] = w1_ref[:, 3 * EMB: 5 * EMB].astype(bf)
        b_col = b1_ref[...] + lax.dot_general(
            w1_ref[:, 5 * EMB: 6 * EMB], qb_ref[...],
            (((1,), (1,)), ((), ())), preferred_element_type=jnp.float32)
        wcat_ref[:, 5 * EMB: 5 * EMB + 1] = b_col.astype(bf)
        wcat_ref[:, 5 * EMB + 1:] = jnp.zeros(
            (wcat_ref.shape[0], EMB - 1), bf)
        hcat_ref[:, 5 * EMB: 5 * EMB + 1] = jnp.ones((bb, 1), bf)
        hcat_ref[:, 5 * EMB + 1:] = jnp.zeros((bb, EMB - 1), bf)

    hcat_ref[:, : 3 * EMB] = x_ref[...].astype(bf)
    hcat_ref[:, 3 * EMB: 4 * EMB] = m_ref[...].astype(bf)
    hcat_ref[:, 4 * EMB: 5 * EMB] = d_ref[...].astype(bf)
    hidden = wcat_ref.shape[0]
    hc = hidden // 2
    hcat = hcat_ref[...]
    out = b2_ref[...]
    for k in range(2):
        acc = lax.dot_general(hcat, wcat_ref[k * hc:(k + 1) * hc, :],
                              (((1,), (1,)), ((), ())),
                              preferred_element_type=jnp.float32)
        hk = jnp.maximum(acc.astype(bf), jnp.bfloat16(0.0))
        out += lax.dot_general(hk, w2_ref[:, k * hc:(k + 1) * hc],
                               (((1,), (1,)), ((), ())),
                               preferred_element_type=jnp.float32)
    out_ref[...] = out


def _mlp_tc(x, m_emb, d_emb, q_w, q_b, w1, b1, w2, b2, n_rows, row_off):
    _, in_dim = x.shape
    hidden = w1.shape[0]
    ncls = w2.shape[0]
    grid = (n_rows // B_BLOCK,)
    off = row_off // B_BLOCK
    return pl.pallas_call(
        _mlp_body,
        grid=grid,
        in_specs=[
            pl.BlockSpec((B_BLOCK, in_dim), lambda i: (i + off, 0)),
            pl.BlockSpec((B_BLOCK, EMB), lambda i: (i, 0)),
            pl.BlockSpec((B_BLOCK, EMB), lambda i: (i, 0)),
            pl.BlockSpec(memory_space=pltpu.VMEM),
            pl.BlockSpec(memory_space=pltpu.VMEM),
            pl.BlockSpec(memory_space=pltpu.VMEM),
            pl.BlockSpec(memory_space=pltpu.VMEM),
            pl.BlockSpec(memory_space=pltpu.VMEM),
            pl.BlockSpec(memory_space=pltpu.VMEM),
        ],
        out_specs=pl.BlockSpec((B_BLOCK, ncls), lambda i: (i, 0)),
        out_shape=jax.ShapeDtypeStruct((n_rows, ncls), jnp.float32),
        scratch_shapes=[
            pltpu.VMEM((B_BLOCK, in_dim + 3 * EMB), jnp.bfloat16),
            pltpu.VMEM((hidden, in_dim + 3 * EMB), jnp.bfloat16),
        ],
        compiler_params=pltpu.CompilerParams(
            dimension_semantics=("arbitrary",)),
    )(x, m_emb, d_emb, q_w, q_b.reshape(1, EMB), w1, b1.reshape(hidden, 1),
      w2, b2.reshape(1, ncls))


def kernel(x, model_ids, decoding_ids, model_emb_w, decoding_emb_w, q_w, q_b,
           w1, b1, w2, b2):
    bf = jnp.bfloat16
    B = x.shape[0]
    n_split = 4
    part = B // n_split
    ids_m = model_ids.astype(jnp.int32)
    ids_d = decoding_ids.astype(jnp.int32)
    embs = [
        _sc_gather(model_emb_w, decoding_emb_w,
                   ids_m[h * part:(h + 1) * part],
                   ids_d[h * part:(h + 1) * part])
        for h in range(n_split)
    ]
    outs = [
        _mlp_tc(x, embs[h][0], embs[h][1], q_w.astype(bf), q_b, w1, b1,
                w2.astype(bf), b2, part, h * part)
        for h in range(n_split)
    ]
    return jnp.concatenate(outs, axis=0)


# speedup vs baseline: 1.0796x; 1.0796x over previous
import functools

import jax
import jax.numpy as jnp
from jax import lax
from jax.experimental import pallas as pl
from jax.experimental.pallas import tpu as pltpu
from jax.experimental.pallas import tpu_sc as plsc

EMB = 128
B_BLOCK = 2048


def _gather_chunk(tab_v, idx_ref, off, buf, chunk):
    def g_body(g, carry):
        rvec = idx_ref[pl.ds(off + g * 16, 16)]
        for j in range(16):
            rid = rvec[j]
            r = g * 16 + j
            for c in range(EMB // 16):
                buf[r, pl.ds(c * 16, 16)] = tab_v[rid, pl.ds(c * 16, 16)]
        return carry
    lax.fori_loop(0, chunk // 16, g_body, 0)


def _sc_gather_body(m_tab_hbm, d_tab_hbm, mi_hbm, di_hbm, om_hbm, od_hbm,
                    m_tab_v, d_tab_v, idx_m, idx_d, buf_a, buf_b, s_a, s_b,
                    *, n_cores, b_per_w):
    chunk = b_per_w // 2
    wid = lax.axis_index("s") * n_cores + lax.axis_index("c")
    base = wid * b_per_w
    pltpu.sync_copy(m_tab_hbm, m_tab_v)
    pltpu.sync_copy(d_tab_hbm, d_tab_v)
    pltpu.sync_copy(mi_hbm.at[pl.ds(base, b_per_w)], idx_m)
    pltpu.sync_copy(di_hbm.at[pl.ds(base, b_per_w)], idx_d)
    _gather_chunk(m_tab_v, idx_m, 0, buf_a, chunk)
    wb_a = pltpu.async_copy(buf_a, om_hbm.at[pl.ds(base, chunk)], s_a)
    _gather_chunk(m_tab_v, idx_m, chunk, buf_b, chunk)
    wb_b = pltpu.async_copy(buf_b, om_hbm.at[pl.ds(base + chunk, chunk)], s_b)
    wb_a.wait()
    _gather_chunk(d_tab_v, idx_d, 0, buf_a, chunk)
    wb_a = pltpu.async_copy(buf_a, od_hbm.at[pl.ds(base, chunk)], s_a)
    wb_b.wait()
    _gather_chunk(d_tab_v, idx_d, chunk, buf_b, chunk)
    wb_b = pltpu.async_copy(buf_b, od_hbm.at[pl.ds(base + chunk, chunk)], s_b)
    wb_a.wait()
    wb_b.wait()


def _sc_gather(model_emb_w, decoding_emb_w, model_ids, decoding_ids):
    B = model_ids.shape[0]
    info = plsc.get_sparse_core_info()
    nw = info.num_cores * info.num_subcores
    b_per_w = B // nw
    mesh = plsc.VectorSubcoreMesh(core_axis_name="c", subcore_axis_name="s")
    k = pl.kernel(
        functools.partial(_sc_gather_body, n_cores=info.num_cores,
                          b_per_w=b_per_w),
        mesh=mesh,
        compiler_params=pltpu.CompilerParams(needs_layout_passes=False),
        out_type=[
            jax.ShapeDtypeStruct((B, EMB), jnp.float32),
            jax.ShapeDtypeStruct((B, EMB), jnp.float32),
        ],
        scratch_types=[
            pltpu.VMEM(model_emb_w.shape, jnp.float32),
            pltpu.VMEM(decoding_emb_w.shape, jnp.float32),
            pltpu.VMEM((b_per_w,), jnp.int32),
            pltpu.VMEM((b_per_w,), jnp.int32),
            pltpu.VMEM((b_per_w // 2, EMB), jnp.float32),
            pltpu.VMEM((b_per_w // 2, EMB), jnp.float32),
            pltpu.SemaphoreType.DMA,
            pltpu.SemaphoreType.DMA,
        ],
    )
    return k(model_emb_w, decoding_emb_w, model_ids, decoding_ids)


def _mlp_body(x_ref, m_ref, d_ref, qw_ref, qb_ref, w1_ref, b1_ref, w2_ref,
              b2_ref, out_ref, hcat_ref, wcat_ref):
    bb = hcat_ref.shape[0]
    bf = jnp.bfloat16

    @pl.when(pl.program_id(0) == 0)
    def _fold():
        wcat_ref[:, : 2 * EMB] = w1_ref[:, : 2 * EMB].astype(bf)
        qfold = lax.dot_general(
            w1_ref[:, 5 * EMB: 6 * EMB].astype(bf), qw_ref[...],
            (((1,), (0,)), ((), ())), preferred_element_type=jnp.float32)
        wcat_ref[:, 2 * EMB: 3 * EMB] = (
            w1_ref[:, 2 * EMB: 3 * EMB] + qfold).astype(bf)
        wcat_ref[:, 3 * EMB: 5 * EMB] = w1_ref[:, 3 * EMB: 5 * EMB].astype(bf)
        b_col = b1_ref[...] + lax.dot_general(
            w1_ref[:, 5 * EMB: 6 * EMB], qb_ref[...],
            (((1,), (1,)), ((), ())), preferred_element_type=jnp.float32)
        wcat_ref[:, 5 * EMB: 5 * EMB + 1] = b_col.astype(bf)
        wcat_ref[:, 5 * EMB + 1:] = jnp.zeros(
            (wcat_ref.shape[0], EMB - 1), bf)
        hcat_ref[:, 5 * EMB: 5 * EMB + 1] = jnp.ones((bb, 1), bf)
        hcat_ref[:, 5 * EMB + 1:] = jnp.zeros((bb, EMB - 1), bf)

    hcat_ref[:, : 3 * EMB] = x_ref[...].astype(bf)
    hcat_ref[:, 3 * EMB: 4 * EMB] = m_ref[...].astype(bf)
    hcat_ref[:, 4 * EMB: 5 * EMB] = d_ref[...].astype(bf)
    hidden = wcat_ref.shape[0]
    hc = hidden // 2
    hcat = hcat_ref[...]
    out = b2_ref[...]
    for k in range(2):
        acc = lax.dot_general(hcat, wcat_ref[k * hc:(k + 1) * hc, :],
                              (((1,), (1,)), ((), ())),
                              preferred_element_type=jnp.float32)
        hk = jnp.maximum(acc.astype(bf), jnp.bfloat16(0.0))
        out += lax.dot_general(hk, w2_ref[:, k * hc:(k + 1) * hc],
                               (((1,), (1,)), ((), ())),
                               preferred_element_type=jnp.float32)
    out_ref[...] = out


def _mlp_tc(x, m_emb, d_emb, q_w, q_b, w1, b1, w2, b2, n_rows, row_off):
    _, in_dim = x.shape
    hidden = w1.shape[0]
    ncls = w2.shape[0]
    grid = (n_rows // B_BLOCK,)
    off = row_off // B_BLOCK
    return pl.pallas_call(
        _mlp_body,
        grid=grid,
        in_specs=[
            pl.BlockSpec((B_BLOCK, in_dim), lambda i: (i + off, 0)),
            pl.BlockSpec((B_BLOCK, EMB), lambda i: (i, 0)),
            pl.BlockSpec((B_BLOCK, EMB), lambda i: (i, 0)),
            pl.BlockSpec(memory_space=pltpu.VMEM),
            pl.BlockSpec(memory_space=pltpu.VMEM),
            pl.BlockSpec(memory_space=pltpu.VMEM),
            pl.BlockSpec(memory_space=pltpu.VMEM),
            pl.BlockSpec(memory_space=pltpu.VMEM),
            pl.BlockSpec(memory_space=pltpu.VMEM),
        ],
        out_specs=pl.BlockSpec((B_BLOCK, ncls), lambda i: (i, 0)),
        out_shape=jax.ShapeDtypeStruct((n_rows, ncls), jnp.float32),
        scratch_shapes=[
            pltpu.VMEM((B_BLOCK, in_dim + 3 * EMB), jnp.bfloat16),
            pltpu.VMEM((hidden, in_dim + 3 * EMB), jnp.bfloat16),
        ],
        compiler_params=pltpu.CompilerParams(
            dimension_semantics=("arbitrary",)),
    )(x, m_emb, d_emb, q_w, q_b.reshape(1, EMB), w1, b1.reshape(hidden, 1),
      w2, b2.reshape(1, ncls))


def kernel(x, model_ids, decoding_ids, model_emb_w, decoding_emb_w, q_w, q_b,
           w1, b1, w2, b2):
    bf = jnp.bfloat16
    B = x.shape[0]
    n_split = 2
    part = B // n_split
    ids_m = model_ids.astype(jnp.int32)
    ids_d = decoding_ids.astype(jnp.int32)
    embs = [
        _sc_gather(model_emb_w, decoding_emb_w,
                   ids_m[h * part:(h + 1) * part],
                   ids_d[h * part:(h + 1) * part])
        for h in range(n_split)
    ]
    outs = [
        _mlp_tc(x, embs[h][0], embs[h][1], q_w.astype(bf), q_b, w1, b1,
                w2.astype(bf), b2, part, h * part)
        for h in range(n_split)
    ]
    return jnp.concatenate(outs, axis=0)
